# baseline (speedup 1.0000x reference)
import jax
import jax.numpy as jnp
from jax import lax
from jax.experimental import pallas as pl
from jax.experimental.pallas import tpu as pltpu


_NCHUNK = 8


def _gather_kernel(ids_ref, vocab_ref, out_ref, stage_ref, tab_ref, sems):
    br, tb, d = out_ref.shape
    n_groups = vocab_ref.shape[0]
    lanes = d // 8

    @pl.when(pl.program_id(1) == 0)
    def _load_and_retile():
        nchunk = _NCHUNK if n_groups % _NCHUNK == 0 else 1
        cg = n_groups // nchunk
        for c in range(nchunk):
            pltpu.make_async_copy(
                vocab_ref.at[pl.ds(c * cg, cg)],
                stage_ref.at[pl.ds(c * cg, cg)],
                sems.at[c],
            ).start()
        unroll = 4
        for c in range(nchunk):
            pltpu.make_async_copy(
                vocab_ref.at[pl.ds(c * cg, cg)],
                stage_ref.at[pl.ds(c * cg, cg)],
                sems.at[c],
            ).wait()
            def body(gg, _, c=c):
                for u in range(unroll):
                    g = c * cg + gg * unroll + u
                    tab_ref[pl.ds(8 * g, 8)] = stage_ref[g].reshape(8, 8, lanes)
                return 0
            lax.fori_loop(0, cg // unroll, body, 0)
            for g in range(c * cg + cg - cg % unroll, c * cg + cg):
                tab_ref[pl.ds(8 * g, 8)] = stage_ref[g].reshape(8, 8, lanes)
        tab_ref[pl.ds(8 * n_groups, 8)] = jnp.zeros((8, 8, lanes), jnp.float32)

    steps = pl.num_programs(1)
    base = (pl.program_id(0) * steps + pl.program_id(1)) * br * tb
    for r in range(br):
        for k in range(tb // 8):
            rows = []
            for t in range(8):
                rows.append(tab_ref[ids_ref[base + r * tb + 8 * k + t]])
            chunk = jnp.stack(rows, axis=0)
            out_ref[r, pl.ds(8 * k, 8), :] = chunk.reshape(8, d)


def kernel(tokens, vocab, mask):
    assert tokens.ndim == 2
    V, D = vocab.shape
    d0, d1 = tokens.shape
    assert d1 % 8 == 0 and D % 8 == 0 and V % 8 == 0

    ids = tokens.reshape(-1).astype(jnp.int32)
    m = mask.reshape(-1)
    ids = jnp.where(m != 0, ids, jnp.int32(V))
    vocab_v = vocab.reshape(V // 8, 8, D)

    br = 1
    for cand in (8, 4, 2):
        if d0 % (2 * cand) == 0:
            br = cand
            break
    steps = d0 // br // 2 if d0 % 2 == 0 else d0 // br
    ncore = d0 // br // steps

    table_bytes = V * D * 4
    tile_bytes = br * d1 * D * 4
    vmem_limit = int(min(64 * 1024 * 1024,
                         2 * table_bytes + 2 * tile_bytes + (8 << 20)))

    out = pl.pallas_call(
        _gather_kernel,
        out_shape=jax.ShapeDtypeStruct((d0, d1, D), vocab.dtype),
        grid_spec=pltpu.PrefetchScalarGridSpec(
            num_scalar_prefetch=1,
            grid=(ncore, steps),
            in_specs=[
                pl.BlockSpec(memory_space=pl.ANY),
            ],
            out_specs=pl.BlockSpec(
                (br, d1, D),
                lambda i, j, ids: (i * pl.num_programs(1) + j, 0, 0)),
            scratch_shapes=[
                pltpu.VMEM((V // 8, 8, D), jnp.float32),
                pltpu.VMEM((V + 8, 8, D // 8), jnp.float32),
                pltpu.SemaphoreType.DMA((_NCHUNK,)),
            ],
        ),
        compiler_params=pltpu.CompilerParams(
            dimension_semantics=("parallel", "arbitrary"),
            vmem_limit_bytes=vmem_limit,
        ),
    )(ids, vocab_v)

    return out


# speedup vs baseline: 1.0490x; 1.0490x over previous
import jax
import jax.numpy as jnp
from jax import lax
from jax.experimental import pallas as pl
from jax.experimental.pallas import tpu as pltpu


def _gather_kernel(ids_ref, vocab_ref, out_ref, tab_ref):
    br, tb, d = out_ref.shape
    n_groups = vocab_ref.shape[0]
    lanes = d // 8

    @pl.when(pl.program_id(1) == 0)
    def _retile():
        unroll = 8
        def body(gg, _):
            for u in range(unroll):
                g = gg * unroll + u
                tab_ref[pl.ds(8 * g, 8)] = vocab_ref[g].reshape(8, 8, lanes)
            return 0
        lax.fori_loop(0, n_groups // unroll, body, 0)
        for g in range(n_groups - n_groups % unroll, n_groups):
            tab_ref[pl.ds(8 * g, 8)] = vocab_ref[g].reshape(8, 8, lanes)
        tab_ref[pl.ds(8 * n_groups, 8)] = jnp.zeros((8, 8, lanes), jnp.float32)

    steps = pl.num_programs(1)
    base = (pl.program_id(0) * steps + pl.program_id(1)) * br * tb
    for r in range(br):
        for k in range(tb // 8):
            rows = []
            for t in range(8):
                rows.append(tab_ref[ids_ref[base + r * tb + 8 * k + t]])
            chunk = jnp.stack(rows, axis=0)
            out_ref[r, pl.ds(8 * k, 8), :] = chunk.reshape(8, d)


def kernel(tokens, vocab, mask):
    assert tokens.ndim == 2
    V, D = vocab.shape
    d0, d1 = tokens.shape
    assert d1 % 8 == 0 and D % 8 == 0 and V % 8 == 0

    ids = tokens.reshape(-1).astype(jnp.int32)
    m = mask.reshape(-1)
    ids = jnp.where(m != 0, ids, jnp.int32(V))
    vocab_v = vocab.reshape(V // 8, 8, D)

    br = 1
    for cand in (8, 4, 2):
        if d0 % (2 * cand) == 0:
            br = cand
            break
    steps = d0 // br // 2 if d0 % 2 == 0 else d0 // br
    ncore = d0 // br // steps

    table_bytes = V * D * 4
    tile_bytes = br * d1 * D * 4
    vmem_limit = int(min(64 * 1024 * 1024,
                         2 * table_bytes + 2 * tile_bytes + (8 << 20)))

    out = pl.pallas_call(
        _gather_kernel,
        out_shape=jax.ShapeDtypeStruct((d0, d1, D), vocab.dtype),
        grid_spec=pltpu.PrefetchScalarGridSpec(
            num_scalar_prefetch=1,
            grid=(ncore, steps),
            in_specs=[
                pl.BlockSpec((V // 8, 8, D), lambda i, j, ids: (0, 0, 0)),
            ],
            out_specs=pl.BlockSpec(
                (br, d1, D),
                lambda i, j, ids: (i * pl.num_programs(1) + j, 0, 0)),
            scratch_shapes=[pltpu.VMEM((V + 8, 8, D // 8), jnp.float32)],
        ),
        compiler_params=pltpu.CompilerParams(
            dimension_semantics=("parallel", "arbitrary"),
            vmem_limit_bytes=vmem_limit,
        ),
    )(ids, vocab_v)

    return out
